# baseline (speedup 1.0000x reference)
import functools

import jax
import jax.numpy as jnp
import numpy as np
from jax import lax
from jax.experimental import pallas as pl
from jax.experimental.pallas import tpu as pltpu
from jax.experimental.pallas import tpu_sc as plsc

_LEVELS = ((100, 100), (50, 50), (25, 25), (13, 13))
_Q = 13294
_H = 8
_D = 32
_LP = 16
_LANES = _H * _LP
_NTILES = 32
_QP = 13312
_QPT = _QP // _NTILES
_QB = 4
_NB = _QPT // _QB
_RPB = _QB * _LANES
_OPB = _QB * _H * _D
_PREP_BQ = 512

_COLPERM = np.stack([np.arange(16), np.arange(16) + 16], axis=1).reshape(-1)


def _prep_body(locx_ref, locy_ref, attw_ref, *out_refs):
    f32 = jnp.float32
    lane = lax.broadcasted_iota(jnp.int32, locx_ref.shape, 1)
    lvl = (lane // 4) % 4
    head = lane // _LP

    def per_level(vals):
        v0, v1, v2, v3 = vals
        return jnp.where(lvl == 0, v0,
               jnp.where(lvl == 1, v1,
               jnp.where(lvl == 2, v2, v3)))

    W = per_level([w for _, w in _LEVELS])
    Hh = per_level([h for h, _ in _LEVELS])
    offs = []
    acc = 0
    for h, w in _LEVELS:
        offs.append(acc)
        acc += h * w
    off = per_level(offs)

    qidx = (pl.program_id(0) * _PREP_BQ
            + lax.broadcasted_iota(jnp.int32, locx_ref.shape, 0))
    qvalid = (qidx < _Q).astype(f32)

    Wf = W.astype(f32)
    Hf = Hh.astype(f32)
    gx = 2.0 * locx_ref[...] - 1.0
    gy = 2.0 * locy_ref[...] - 1.0
    ix = ((gx + 1.0) * Wf - 1.0) * 0.5
    iy = ((gy + 1.0) * Hf - 1.0) * 0.5
    x0f = jnp.floor(ix)
    y0f = jnp.floor(iy)
    wx1 = ix - x0f
    wx0 = 1.0 - wx1
    wy1 = iy - y0f
    wy0 = 1.0 - wy1
    x0 = x0f.astype(jnp.int32)
    y0 = y0f.astype(jnp.int32)
    x1 = x0 + 1
    y1 = y0 + 1
    vx0 = ((x0 >= 0) & (x0 <= W - 1)).astype(f32)
    vx1 = ((x1 >= 0) & (x1 <= W - 1)).astype(f32)
    vy0 = ((y0 >= 0) & (y0 <= Hh - 1)).astype(f32)
    vy1 = ((y1 >= 0) & (y1 <= Hh - 1)).astype(f32)
    xc0 = jnp.clip(x0, 0, W - 1)
    xc1 = jnp.clip(x1, 0, W - 1)
    yc0 = jnp.clip(y0, 0, Hh - 1)
    yc1 = jnp.clip(y1, 0, Hh - 1)
    attw = attw_ref[...] * qvalid

    corners = [
        (xc0, yc0, wx0 * wy0 * vx0 * vy0),
        (xc1, yc0, wx1 * wy0 * vx1 * vy0),
        (xc0, yc1, wx0 * wy1 * vx0 * vy1),
        (xc1, yc1, wx1 * wy1 * vx1 * vy1),
    ]
    for c, (xc, yc, wgt) in enumerate(corners):
        out_refs[c][...] = (off + yc * W + xc) * _H + head
        out_refs[4 + c][...] = attw * wgt


def _prep(locx, locy, attw):
    grid = (_QP // _PREP_BQ,)
    spec = pl.BlockSpec((_PREP_BQ, _LANES), lambda i: (i, 0))
    return pl.pallas_call(
        _prep_body,
        grid=grid,
        in_specs=[spec, spec, spec],
        out_specs=[spec] * 8,
        out_shape=(
            [jax.ShapeDtypeStruct((_QP, _LANES), jnp.int32)] * 4
            + [jax.ShapeDtypeStruct((_QP, _LANES), jnp.float32)] * 4
        ),
    )(locx, locy, attw)


def _sc_body(table, i0, i1, i2, i3, w0, w1, w2, w3, out_hbm, *scratch):
    idxv = (scratch[0:4], scratch[4:8])
    wv = (scratch[8:12], scratch[12:16])
    rows = (scratch[16:20], scratch[20:24])
    outv = scratch[24:26]
    sem_g = scratch[26:28]
    sem_io = scratch[28:30]
    sem_o = scratch[30:32]
    idx_hbm = (i0, i1, i2, i3)
    w_hbm = (w0, w1, w2, w3)

    wid = lax.axis_index("s") * 2 + lax.axis_index("c")
    qbase = wid * _QPT

    def io_copies(b, slot):
        q0 = qbase + b * _QB
        cps = []
        for c in range(4):
            cps.append(pltpu.make_async_copy(
                idx_hbm[c].at[pl.ds(q0, _QB)], idxv[slot][c], sem_io[slot]))
            cps.append(pltpu.make_async_copy(
                w_hbm[c].at[pl.ds(q0, _QB)], wv[slot][c], sem_io[slot]))
        return cps

    def gather_copies(slot):
        return [pltpu.make_async_copy(table.at[idxv[slot][c].at[qi]],
                                      rows[slot][c].at[qi], sem_g[slot])
                for c in range(4) for qi in range(_QB)]

    def out_copy(b, slot):
        off = (qbase + b * _QB) * (_H * _D)
        return pltpu.make_async_copy(outv[slot],
                                     out_hbm.at[pl.ds(off, _OPB)],
                                     sem_o[slot])

    def compute(slot):
        ov = outv[slot]
        zero = jnp.zeros((16,), jnp.float32)
        for qi in range(_QB):
            def h_body(h, _, qi=qi):
                a0 = [zero] * 4
                a1 = [zero] * 4
                for c in range(4):
                    base = h * _LP
                    wvec = wv[slot][c][qi, pl.ds(base, _LP)]
                    rws = rows[slot][c]
                    for lp in range(_LP):
                        r0, r1 = plsc.unpack(
                            rws[qi, base + lp, :],
                            format=plsc.PackFormat.INTERLEAVED)
                        wgt = wvec[lp]
                        a0[c] = a0[c] + wgt * r0
                        a1[c] = a1[c] + wgt * r1
                o = qi * (_H * _D) + h * _D
                ov[pl.ds(o, 16)] = (a0[0] + a0[1]) + (a0[2] + a0[3])
                ov[pl.ds(o + 16, 16)] = (a1[0] + a1[1]) + (a1[2] + a1[3])
                return 0

            lax.fori_loop(0, _H, h_body, 0)

    for cp in io_copies(0, 0):
        cp.start()
    for cp in io_copies(0, 0):
        cp.wait()
    for cp in gather_copies(0):
        cp.start()
    for cp in io_copies(1, 1):
        cp.start()

    def pair_body(bp, _):
        for sub in range(2):
            b = bp * 2 + sub
            s = sub
            o = 1 - sub

            @pl.when(b + 1 < _NB)
            def _():
                for cp in io_copies(b + 1, o):
                    cp.wait()
                for cp in gather_copies(o):
                    cp.start()

            for cp in gather_copies(s):
                cp.wait()

            @pl.when(b >= 2)
            def _():
                out_copy(b - 2, s).wait()

            compute(s)
            out_copy(b, s).start()

            @pl.when(b + 2 < _NB)
            def _():
                for cp in io_copies(b + 2, s):
                    cp.start()
        return 0

    lax.fori_loop(0, _NB // 2, pair_body, 0)
    out_copy(_NB - 2, 0).wait()
    out_copy(_NB - 1, 1).wait()


def _sc_gather(table, idx_list, w_list):
    mesh = plsc.VectorSubcoreMesh(core_axis_name="c", subcore_axis_name="s")
    scratch = (
        [pltpu.VMEM((_QB, _LANES), jnp.int32)] * 8
        + [pltpu.VMEM((_QB, _LANES), jnp.float32)] * 8
        + [pltpu.VMEM((_QB, _LANES, _D), jnp.bfloat16)] * 8
        + [pltpu.VMEM((_OPB,), jnp.float32)] * 2
        + [pltpu.SemaphoreType.DMA] * 6
    )
    return pl.kernel(
        _sc_body,
        out_type=jax.ShapeDtypeStruct((_QP * _H * _D,), jnp.float32),
        mesh=mesh,
        compiler_params=pltpu.CompilerParams(use_tc_tiling_on_sc=False,
                                             needs_layout_passes=False),
        scratch_types=scratch,
    )(table, *idx_list, *w_list)


def kernel(value, value_spatial_shapes, sampling_locations, attention_weights):
    del value_spatial_shapes
    locs = sampling_locations.reshape(_Q, _LANES, 2)
    locx = locs[..., 0]
    locy = locs[..., 1]
    attw = attention_weights.reshape(_Q, _LANES)
    i0, i1, i2, i3, w0, w1, w2, w3 = _prep(locx, locy, attw)
    table = value.reshape(_Q * _H, _D)[:, _COLPERM].astype(jnp.bfloat16)
    out = _sc_gather(table, (i0, i1, i2, i3), (w0, w1, w2, w3))
    return out[:_Q * _H * _D].reshape(1, _Q, _H * _D)


# speedup vs baseline: 1.0220x; 1.0220x over previous
import functools

import jax
import jax.numpy as jnp
import numpy as np
from jax import lax
from jax.experimental import pallas as pl
from jax.experimental.pallas import tpu as pltpu
from jax.experimental.pallas import tpu_sc as plsc

_LEVELS = ((100, 100), (50, 50), (25, 25), (13, 13))
_Q = 13294
_H = 8
_D = 32
_LP = 16
_LANES = _H * _LP
_NTILES = 32
_QP = 13312
_QPT = _QP // _NTILES
_QB = 4
_NB = _QPT // _QB
_RPB = _QB * _LANES
_OPB = _QB * _H * _D
_PREP_BQ = 512

_COLPERM = np.stack([np.arange(16), np.arange(16) + 16], axis=1).reshape(-1)


def _prep_body(locx_ref, locy_ref, attw_ref, *out_refs):
    f32 = jnp.float32
    locx = locx_ref[...]
    locy = locy_ref[...]
    lane = lax.broadcasted_iota(jnp.int32, locx.shape, 1)
    lvl = (lane // 4) % 4
    head = lane // _LP

    def per_level(vals):
        v0, v1, v2, v3 = vals
        return jnp.where(lvl == 0, v0,
               jnp.where(lvl == 1, v1,
               jnp.where(lvl == 2, v2, v3)))

    W = per_level([w for _, w in _LEVELS])
    Hh = per_level([h for h, _ in _LEVELS])
    offs = []
    acc = 0
    for h, w in _LEVELS:
        offs.append(acc)
        acc += h * w
    off = per_level(offs)

    qidx = (pl.program_id(0) * _PREP_BQ
            + lax.broadcasted_iota(jnp.int32, locx.shape, 0))
    qvalid = (qidx < _Q).astype(f32)

    Wf = W.astype(f32)
    Hf = Hh.astype(f32)
    gx = 2.0 * locx - 1.0
    gy = 2.0 * locy - 1.0
    ix = ((gx + 1.0) * Wf - 1.0) * 0.5
    iy = ((gy + 1.0) * Hf - 1.0) * 0.5
    x0f = jnp.floor(ix)
    y0f = jnp.floor(iy)
    wx1 = ix - x0f
    wx0 = 1.0 - wx1
    wy1 = iy - y0f
    wy0 = 1.0 - wy1
    x0 = x0f.astype(jnp.int32)
    y0 = y0f.astype(jnp.int32)
    x1 = x0 + 1
    y1 = y0 + 1
    vx0 = ((x0 >= 0) & (x0 <= W - 1)).astype(f32)
    vx1 = ((x1 >= 0) & (x1 <= W - 1)).astype(f32)
    vy0 = ((y0 >= 0) & (y0 <= Hh - 1)).astype(f32)
    vy1 = ((y1 >= 0) & (y1 <= Hh - 1)).astype(f32)
    xc0 = jnp.clip(x0, 0, W - 1)
    xc1 = jnp.clip(x1, 0, W - 1)
    yc0 = jnp.clip(y0, 0, Hh - 1)
    yc1 = jnp.clip(y1, 0, Hh - 1)
    attw = attw_ref[...] * qvalid

    corners = [
        (xc0, yc0, wx0 * wy0 * vx0 * vy0),
        (xc1, yc0, wx1 * wy0 * vx1 * vy0),
        (xc0, yc1, wx0 * wy1 * vx0 * vy1),
        (xc1, yc1, wx1 * wy1 * vx1 * vy1),
    ]
    for c, (xc, yc, wgt) in enumerate(corners):
        out_refs[c][...] = (off + yc * W + xc) * _H + head
        out_refs[4 + c][...] = attw * wgt


def _prep(locx, locy, attw):
    grid = (_QP // _PREP_BQ,)
    spec = pl.BlockSpec((_PREP_BQ, _LANES), lambda i: (i, 0))
    return pl.pallas_call(
        _prep_body,
        grid=grid,
        in_specs=[spec, spec, spec],
        out_specs=[spec] * 8,
        out_shape=(
            [jax.ShapeDtypeStruct((_QP, _LANES), jnp.int32)] * 4
            + [jax.ShapeDtypeStruct((_QP, _LANES), jnp.float32)] * 4
        ),
    )(locx, locy, attw)


def _sc_body(table, i0, i1, i2, i3, w0, w1, w2, w3, out_hbm, *scratch):
    idxv = (scratch[0:4], scratch[4:8])
    wv = (scratch[8:12], scratch[12:16])
    rows = (scratch[16:20], scratch[20:24])
    outv = scratch[24:26]
    sem_g = scratch[26:28]
    sem_io = scratch[28:30]
    sem_o = scratch[30:32]
    idx_hbm = (i0, i1, i2, i3)
    w_hbm = (w0, w1, w2, w3)

    wid = lax.axis_index("s") * 2 + lax.axis_index("c")
    qbase = wid * _QPT

    def io_copies(b, slot):
        off = (qbase + b * _QB) * _LANES
        cps = []
        for c in range(4):
            cps.append(pltpu.make_async_copy(
                idx_hbm[c].at[pl.ds(off, _RPB)], idxv[slot][c], sem_io[slot]))
            cps.append(pltpu.make_async_copy(
                w_hbm[c].at[pl.ds(off, _RPB)], wv[slot][c], sem_io[slot]))
        return cps

    def gather_copies(slot):
        return [pltpu.make_async_copy(table.at[idxv[slot][c]], rows[slot][c],
                                      sem_g[slot]) for c in range(4)]

    def out_copy(b, slot):
        off = (qbase + b * _QB) * (_H * _D)
        return pltpu.make_async_copy(outv[slot],
                                     out_hbm.at[pl.ds(off, _OPB)],
                                     sem_o[slot])

    def compute(slot):
        ov = outv[slot]
        zero = jnp.zeros((16,), jnp.float32)
        for qi in range(_QB):
            def h_body(h, _, qi=qi):
                a0 = [zero] * 4
                a1 = [zero] * 4
                for c in range(4):
                    base = qi * _LANES + h * _LP
                    wvec = wv[slot][c][pl.ds(base, _LP)]
                    rws = rows[slot][c]
                    for lp in range(_LP):
                        r0, r1 = plsc.unpack(
                            rws[base + lp, :],
                            format=plsc.PackFormat.INTERLEAVED)
                        wgt = wvec[lp]
                        a0[c] = a0[c] + wgt * r0
                        a1[c] = a1[c] + wgt * r1
                o = qi * (_H * _D) + h * _D
                ov[pl.ds(o, 16)] = (a0[0] + a0[1]) + (a0[2] + a0[3])
                ov[pl.ds(o + 16, 16)] = (a1[0] + a1[1]) + (a1[2] + a1[3])
                return 0

            lax.fori_loop(0, _H, h_body, 0)

    for cp in io_copies(0, 0):
        cp.start()
    for cp in io_copies(0, 0):
        cp.wait()
    for cp in gather_copies(0):
        cp.start()
    for cp in io_copies(1, 1):
        cp.start()

    def pair_body(bp, _):
        for sub in range(2):
            b = bp * 2 + sub
            s = sub
            o = 1 - sub

            @pl.when(b + 1 < _NB)
            def _():
                for cp in io_copies(b + 1, o):
                    cp.wait()
                for cp in gather_copies(o):
                    cp.start()

            for cp in gather_copies(s):
                cp.wait()

            @pl.when(b >= 2)
            def _():
                out_copy(b - 2, s).wait()

            compute(s)
            out_copy(b, s).start()

            @pl.when(b + 2 < _NB)
            def _():
                for cp in io_copies(b + 2, s):
                    cp.start()
        return 0

    lax.fori_loop(0, _NB // 2, pair_body, 0)
    out_copy(_NB - 2, 0).wait()
    out_copy(_NB - 1, 1).wait()


def _sc_gather(table, idx_list, w_list):
    mesh = plsc.VectorSubcoreMesh(core_axis_name="c", subcore_axis_name="s")
    scratch = (
        [pltpu.VMEM((_RPB,), jnp.int32)] * 8
        + [pltpu.VMEM((_RPB,), jnp.float32)] * 8
        + [pltpu.VMEM((_RPB, _D), jnp.bfloat16)] * 8
        + [pltpu.VMEM((_OPB,), jnp.float32)] * 2
        + [pltpu.SemaphoreType.DMA] * 6
    )
    return pl.kernel(
        _sc_body,
        out_type=jax.ShapeDtypeStruct((_QP * _H * _D,), jnp.float32),
        mesh=mesh,
        compiler_params=pltpu.CompilerParams(use_tc_tiling_on_sc=False,
                                             needs_layout_passes=False),
        scratch_types=scratch,
    )(table, *idx_list, *w_list)


def kernel(value, value_spatial_shapes, sampling_locations, attention_weights):
    del value_spatial_shapes
    locs = sampling_locations.reshape(_Q, _LANES, 2)
    locx = locs[..., 0]
    locy = locs[..., 1]
    attw = attention_weights.reshape(_Q, _LANES)
    i0, i1, i2, i3, w0, w1, w2, w3 = _prep(locx, locy, attw)
    table = value.reshape(_Q * _H, _D)[:, _COLPERM].astype(jnp.bfloat16)
    out = _sc_gather(table,
                     [x.reshape(-1) for x in (i0, i1, i2, i3)],
                     [x.reshape(-1) for x in (w0, w1, w2, w3)])
    return out[:_Q * _H * _D].reshape(1, _Q, _H * _D)
